# baseline (speedup 1.0000x reference)
import jax
import jax.numpy as jnp
from jax.experimental import pallas as pl

_NSEL = 300


def _int_key(x):
    s = jax.lax.bitcast_convert_type(x, jnp.int32)
    return jnp.where(s >= 0, s, s ^ jnp.int32(0x7FFFFFFF))


def _rank_threshold(keys, idx, k, idx_bits):
    sign = jnp.int32(-2147483648)

    def body_k(i, u):
        u2 = u | (jnp.int32(1) << (31 - i))
        cnt = jnp.sum((keys >= (u2 ^ sign)).astype(jnp.int32))
        return jnp.where(cnt >= k, u2, u)

    tk = jax.lax.fori_loop(0, 32, body_k, jnp.int32(0)) ^ sign
    quota = k - jnp.sum((keys > tk).astype(jnp.int32))

    def body_i(i, t):
        b = idx_bits - 1 - i
        t2 = t + (jnp.int32(1) << b)
        cnt = jnp.sum(((keys == tk) & (idx <= t2)).astype(jnp.int32))
        return jnp.where(cnt < quota, t2, t)

    ti = jax.lax.fori_loop(0, idx_bits, body_i, jnp.int32(-1)) + 1
    return tk, ti


def _stage1_kernel(x_ref, keys_ref, thr_ref):
    x = x_ref[...]
    key = _int_key(jnp.max(x, axis=2))
    keys_ref[0] = key
    n = key.shape[1]
    ridx = jax.lax.broadcasted_iota(jnp.int32, (1, n), 1)
    tk, ti = _rank_threshold(key, ridx, _NSEL, 15)
    lane = jax.lax.broadcasted_iota(jnp.int32, (1, 128), 1)
    thr_ref[0] = jnp.where(lane == 0, tk, ti)


def _stage2_kernel(c_ref, ckeys_ref, thr_ref):
    c = c_ref[...]
    _, s, cc = c.shape
    key = _int_key(c)
    ckeys_ref[...] = key
    pos = (jax.lax.broadcasted_iota(jnp.int32, (1, s, cc), 1) * cc
           + jax.lax.broadcasted_iota(jnp.int32, (1, s, cc), 2))
    tk, ti = _rank_threshold(key, pos, _NSEL, 15)
    lane = jax.lax.broadcasted_iota(jnp.int32, (1, 128), 1)
    thr_ref[0] = jnp.where(lane == 0, tk, ti)


def _mask_to_indices(mask):
    def one(m):
        return jnp.nonzero(m, size=_NSEL, fill_value=0)[0].astype(jnp.int32)

    return jax.vmap(one)(mask)


def kernel(pred_logits, pred_boxes, target_sizes):
    B, N, C = pred_logits.shape

    keys, thr1 = pl.pallas_call(
        _stage1_kernel,
        grid=(B,),
        in_specs=[pl.BlockSpec((1, N, C), lambda b: (b, 0, 0))],
        out_specs=[pl.BlockSpec((1, 1, N), lambda b: (b, 0, 0)),
                   pl.BlockSpec((1, 1, 128), lambda b: (b, 0, 0))],
        out_shape=[jax.ShapeDtypeStruct((B, 1, N), jnp.int32),
                   jax.ShapeDtypeStruct((B, 1, 128), jnp.int32)],
    )(pred_logits)
    keys = keys.reshape(B, N)
    tk1 = thr1[:, 0, 0:1]
    ti1 = thr1[:, 0, 1:2]

    ridx = jnp.arange(N, dtype=jnp.int32)[None, :]
    mask1 = (keys > tk1) | ((keys == tk1) & (ridx <= ti1))
    rows = _mask_to_indices(mask1)

    cand = jnp.take_along_axis(pred_logits, rows[:, :, None], axis=1)

    ckeys, thr2 = pl.pallas_call(
        _stage2_kernel,
        grid=(B,),
        in_specs=[pl.BlockSpec((1, _NSEL, C), lambda b: (b, 0, 0))],
        out_specs=[pl.BlockSpec((1, _NSEL, C), lambda b: (b, 0, 0)),
                   pl.BlockSpec((1, 1, 128), lambda b: (b, 0, 0))],
        out_shape=[jax.ShapeDtypeStruct((B, _NSEL, C), jnp.int32),
                   jax.ShapeDtypeStruct((B, 1, 128), jnp.int32)],
    )(cand)
    M = _NSEL * C
    ckeys = ckeys.reshape(B, M)
    tk2 = thr2[:, 0, 0:1]
    tp2 = thr2[:, 0, 1:2]

    pidx = jnp.arange(M, dtype=jnp.int32)[None, :]
    mask2 = (ckeys > tk2) | ((ckeys == tk2) & (pidx <= tp2))
    psel = _mask_to_indices(mask2)

    vals = jnp.take_along_axis(cand.reshape(B, M), psel, axis=1)
    svals, order = jax.lax.top_k(vals, _NSEL)
    psel = jnp.take_along_axis(psel, order, axis=1)
    labels = psel % C
    win_rows = jnp.take_along_axis(rows, psel // C, axis=1)

    bsel = jnp.take_along_axis(pred_boxes, win_rows[:, :, None], axis=1)
    cx, cy, w, h = bsel[..., 0], bsel[..., 1], bsel[..., 2], bsel[..., 3]
    xyxy = jnp.stack([cx - w * 0.5, cy - h * 0.5, cx + w * 0.5, cy + h * 0.5],
                     axis=-1)
    img_h = target_sizes[:, 0].astype(jnp.float32)
    img_w = target_sizes[:, 1].astype(jnp.float32)
    scale = jnp.stack([img_w, img_h, img_w, img_h], axis=1)
    return jax.nn.sigmoid(svals), labels, xyxy * scale[:, None, :]


# speedup vs baseline: 2.7530x; 2.7530x over previous
import jax
import jax.numpy as jnp
from jax.experimental import pallas as pl

_NSEL = 300


def _rowmax_kernel(x_ref, out_ref):
    x = x_ref[...]
    m = jnp.max(x, axis=2)
    s = jax.lax.bitcast_convert_type(m, jnp.int32)
    key = jnp.where(s >= 0, s, s ^ jnp.int32(0x7FFFFFFF))
    out_ref[0] = key


def kernel(pred_logits, pred_boxes, target_sizes):
    B, N, C = pred_logits.shape
    keys = pl.pallas_call(
        _rowmax_kernel,
        grid=(B,),
        in_specs=[pl.BlockSpec((1, N, C), lambda b: (b, 0, 0))],
        out_specs=pl.BlockSpec((1, 1, N), lambda b: (b, 0, 0)),
        out_shape=jax.ShapeDtypeStruct((B, 1, N), jnp.int32),
    )(pred_logits)
    keys = keys.reshape(B, N)

    _, rows = jax.lax.top_k(keys, _NSEL)
    rows = jnp.sort(rows, axis=1)

    cand = jnp.take_along_axis(pred_logits, rows[:, :, None], axis=1)
    cvals, cpos = jax.lax.top_k(cand.reshape(B, _NSEL * C), _NSEL)
    j = cpos // C
    labels = cpos % C
    win_rows = jnp.take_along_axis(rows, j, axis=1)

    bsel = jnp.take_along_axis(pred_boxes, win_rows[:, :, None], axis=1)
    cx, cy, w, h = bsel[..., 0], bsel[..., 1], bsel[..., 2], bsel[..., 3]
    xyxy = jnp.stack([cx - w * 0.5, cy - h * 0.5, cx + w * 0.5, cy + h * 0.5],
                     axis=-1)
    img_h = target_sizes[:, 0].astype(jnp.float32)
    img_w = target_sizes[:, 1].astype(jnp.float32)
    scale = jnp.stack([img_w, img_h, img_w, img_h], axis=1)
    scores = jax.nn.sigmoid(cvals)
    return scores, labels, xyxy * scale[:, None, :]
